# baseline (speedup 1.0000x reference)
import functools

import jax
import jax.numpy as jnp
from jax import lax
from jax.experimental import pallas as pl
from jax.experimental.pallas import tpu as pltpu
from jax.experimental.pallas import tpu_sc as plsc

_K = 8192
_D = 256
_BIG = float(3e38)
_SEG = _K // 16

_mesh1 = plsc.VectorSubcoreMesh(core_axis_name="c", subcore_axis_name="s",
                                num_cores=1, num_subcores=16)


def _dist_body(z_ref, cb_ref, out_ref):
    z = z_ref[...]
    cb = cb_ref[...]
    d = cb - z[None, :]
    out_ref[...] = jnp.sum(d * d, axis=1)


def _dists_tc(z_flat, codebook):
    return pl.pallas_call(
        _dist_body,
        grid=(8,),
        in_specs=[
            pl.BlockSpec((_D,), lambda i: (0,)),
            pl.BlockSpec((_K // 8, _D), lambda i: (i, 0)),
        ],
        out_specs=pl.BlockSpec((_K // 8,), lambda i: (i,)),
        out_shape=jax.ShapeDtypeStruct((_K,), jnp.float32),
    )(z_flat, codebook)


@functools.partial(
    pl.kernel,
    out_type=jax.ShapeDtypeStruct((16,), jnp.int32),
    mesh=_mesh1,
    compiler_params=pltpu.CompilerParams(needs_layout_passes=False,
                                         use_tc_tiling_on_sc=True),
    scratch_types=[
        pltpu.VMEM((_SEG,), jnp.float32),
        pltpu.VMEM((16,), jnp.float32),
        pltpu.VMEM((16,), jnp.int32),
        pltpu.VMEM((16,), jnp.int32),
        pltpu.VMEM((256,), jnp.float32),
        pltpu.VMEM((256,), jnp.int32),
        pltpu.VMEM((16,), jnp.int32),
        pltpu.VMEM((16,), jnp.float32),
        pltpu.VMEM((48, _SEG), jnp.float32),
        pltpu.VMEM((16,), jnp.float32),
        pltpu.VMEM((256,), jnp.float32),
        pltpu.VMEM((16,), jnp.int32),
        pltpu.VMEM_SHARED((256,), jnp.float32),
        pltpu.VMEM_SHARED((256,), jnp.int32),
        pltpu.VMEM_SHARED((16,), jnp.int32),
        pltpu.VMEM_SHARED((16,), jnp.float32),
        pltpu.VMEM_SHARED((256,), jnp.float32),
    ],
)
def _tail_sc(dists_hbm, cur_hbm, adj_hbm, out_hbm,
             dv, td_v, ti_v, cur_v, md_v, mi_v, cv_v, cd_v, band_v, pw_v,
             pa_v, ov_v, sh_d5, sh_i5, sh_cand, sh_cd, sh_part):
    s = lax.axis_index("s")
    lane = lax.iota(jnp.int32, 16)

    pltpu.sync_copy(dists_hbm.at[pl.ds(s * _SEG, _SEG)], dv)
    m = jnp.full((16,), _BIG, jnp.float32)
    mi = jnp.zeros((16,), jnp.int32)
    base = s * _SEG + lane
    for k in range(_SEG // 16):
        d = dv[pl.ds(k * 16, 16)]
        take = d < m
        m = jnp.where(take, d, m)
        mi = jnp.where(take, base + k * 16, mi)
    d5 = jnp.full((16,), _BIG, jnp.float32)
    i5 = jnp.zeros((16,), jnp.int32)
    for i in range(5):
        mn = jnp.min(m)
        cand = jnp.min(jnp.where(m == mn, mi, jnp.int32(_K)))
        d5 = jnp.where(lane == i, jnp.broadcast_to(mn, (16,)), d5)
        i5 = jnp.where(lane == i, jnp.broadcast_to(cand, (16,)), i5)
        m = jnp.where(mi == cand, _BIG, m)
    td_v[...] = d5
    ti_v[...] = i5
    pltpu.sync_copy(td_v, sh_d5.at[pl.ds(s * 16, 16)])
    pltpu.sync_copy(ti_v, sh_i5.at[pl.ds(s * 16, 16)])

    plsc.subcore_barrier()

    @pl.when(s == 0)
    def _merge():
        pltpu.sync_copy(sh_d5, md_v)
        pltpu.sync_copy(sh_i5, mi_v)
        pltpu.sync_copy(cur_hbm, cur_v)
        mm = jnp.full((16,), _BIG, jnp.float32)
        mmi = jnp.full((16,), _K, jnp.int32)
        for r in range(16):
            d = md_v[pl.ds(r * 16, 16)]
            ii = mi_v[pl.ds(r * 16, 16)]
            take = (d < mm) | ((d == mm) & (ii < mmi))
            mm = jnp.where(take, d, mm)
            mmi = jnp.where(take, ii, mmi)
        curv = cur_v[...]
        candvec = curv
        cdvec = jnp.full((16,), _BIG, jnp.float32)
        for i in range(5):
            mn = jnp.min(mm)
            cand = jnp.min(jnp.where(mm == mn, mmi, jnp.int32(_K)))
            candvec = jnp.where(lane == i, jnp.broadcast_to(cand, (16,)),
                                candvec)
            cdvec = jnp.where(lane == i, jnp.broadcast_to(mn, (16,)), cdvec)
            mm = jnp.where(mmi == cand, _BIG, mm)
        cv_v[...] = candvec
        cd_v[...] = cdvec
        pltpu.sync_copy(cv_v, sh_cand)
        pltpu.sync_copy(cd_v, sh_cd)

    plsc.subcore_barrier()

    pltpu.sync_copy(sh_cand, cv_v)
    candv = cv_v[...]
    subl = []
    for r in range(6):
        row = candv[r]
        rb = pl.multiple_of((row >> 3) << 3, 8)
        cb0 = pl.multiple_of(s * _SEG, _SEG)
        subl.append(row & 7)
        pltpu.sync_copy(adj_hbm.at[pl.ds(rb, 8), pl.ds(cb0, _SEG)],
                        band_v.at[pl.ds(r * 8, 8), :])
    parts = jnp.zeros((16,), jnp.float32)
    accs = [jnp.zeros((16,), jnp.float32) for _ in range(5)]
    for k in range(_SEG // 16):
        b = band_v[5 * 8 + subl[5], pl.ds(k * 16, 16)]
        for ci in range(5):
            a = band_v[ci * 8 + subl[ci], pl.ds(k * 16, 16)]
            accs[ci] = accs[ci] + jnp.abs(a - b)
    for ci in range(5):
        ssum = jnp.sum(accs[ci])
        parts = jnp.where(lane == ci, jnp.broadcast_to(ssum, (16,)), parts)
    pw_v[...] = parts
    pltpu.sync_copy(pw_v, sh_part.at[pl.ds(s * 16, 16)])

    plsc.subcore_barrier()

    @pl.when(s == 0)
    def _finish():
        pltpu.sync_copy(sh_part, pa_v)
        tot = jnp.zeros((16,), jnp.float32)
        for ww in range(16):
            tot = tot + pa_v[pl.ds(ww * 16, 16)]
        gd = tot * jnp.float32(1.0 / _K)
        cdvec = cd_v[...]
        candvec = cv_v[...]
        curv = cur_v[...]
        svec = -cdvec + jnp.float32(0.1) * gd
        svec = jnp.where(candvec == curv, -_BIG, svec)
        svec = jnp.where(lane < 5, svec, -_BIG)
        mx = jnp.max(svec)
        lf = jnp.min(jnp.where(svec == mx, lane, jnp.int32(16)))
        best = jnp.max(jnp.where(lane == lf, candvec, jnp.int32(-1)))
        ov_v[...] = jnp.broadcast_to(best, (16,))
        pltpu.sync_copy(ov_v, out_hbm)


@functools.partial(
    pl.kernel,
    out_type=jax.ShapeDtypeStruct((16,), jnp.int32),
    mesh=_mesh1,
    compiler_params=pltpu.CompilerParams(needs_layout_passes=False,
                                         use_tc_tiling_on_sc=True),
    scratch_types=[pltpu.VMEM((16,), jnp.int32)],
)
def _trivial_sc(cur_hbm, out_hbm, t_v):
    s = lax.axis_index("s")

    @pl.when(s == 0)
    def _go():
        pltpu.sync_copy(cur_hbm, t_v)
        pltpu.sync_copy(t_v, out_hbm)


def kernel(z_flat, codebook, adjacency, current_sym):
    cur16 = jnp.full((16,), current_sym, dtype=jnp.int32)
    dists = _dists_tc(z_flat, codebook)
    out = _trivial_sc(cur16 + dists[0:16].astype(jnp.int32))
    return out[0]


# speedup vs baseline: 1.4354x; 1.4354x over previous
import jax
import jax.numpy as jnp
from jax import lax
from jax.experimental import pallas as pl
from jax.experimental.pallas import tpu as pltpu

_K = 8192
_D = 256
_NB = 16
_BLK = _K // _NB
_NEG = float(-3e38)
_BIG = float(3e38)


def _body(z_ref, cb_ref, cur_ref, adj_ref, out_ref, dist_ref, rows_ref, sem):
    i = pl.program_id(0)
    z = z_ref[...]
    cb = cb_ref[...]
    z2 = z.reshape(_D, 1)
    ones2 = jnp.ones((_D, 1), jnp.float32)
    a = lax.dot_general(cb, z2, (((1,), (0,)), ((), ())),
                        preferred_element_type=jnp.float32)
    b = lax.dot_general(cb * cb, ones2, (((1,), (0,)), ((), ())),
                        preferred_element_type=jnp.float32)
    dist_ref[pl.ds(i * _BLK, _BLK)] = (b - 2.0 * a).reshape(_BLK)

    @pl.when(i == _NB - 1)
    def _tail():
        d2 = dist_ref[...].reshape(64, 128)
        iota2 = lax.broadcasted_iota(jnp.int32, (64, 128), 0) * 128 + \
            lax.broadcasted_iota(jnp.int32, (64, 128), 1)

        cands = []
        cand_dists = []
        for _ in range(5):
            mn = jnp.min(d2)
            idx = jnp.min(jnp.where(d2 == mn, iota2, jnp.int32(_K)))
            cands.append(idx)
            cand_dists.append(mn)
            d2 = jnp.where(iota2 == idx, _BIG, d2)

        cur = cur_ref[0]
        copies = []
        for r in range(5):
            copies.append(pltpu.make_async_copy(
                adj_ref.at[pl.ds(cands[r], 1)], rows_ref.at[pl.ds(r, 1)],
                sem))
        copies.append(pltpu.make_async_copy(
            adj_ref.at[pl.ds(cur, 1)], rows_ref.at[pl.ds(5, 1)], sem))
        for cpy in copies:
            cpy.start()
        for cpy in copies:
            cpy.wait()

        rows = rows_ref[...]
        gdiff = jnp.mean(jnp.abs(rows[:5, :] - rows[5:6, :]), axis=1)

        best_score = jnp.full((), _NEG, jnp.float32)
        best_s = jnp.int32(0)
        for r in range(5):
            sc = -cand_dists[r] + 0.1 * gdiff[r]
            sc = jnp.where(cands[r] == cur, _NEG, sc)
            take = sc > best_score
            best_score = jnp.where(take, sc, best_score)
            best_s = jnp.where(take, cands[r], best_s)
        out_ref[0] = best_s


@jax.jit
def _run(z_flat, codebook, adjacency, cur_arr):
    out = pl.pallas_call(
        _body,
        grid=(_NB,),
        in_specs=[
            pl.BlockSpec((_D,), lambda i: (0,)),
            pl.BlockSpec((_BLK, _D), lambda i: (i, 0)),
            pl.BlockSpec(memory_space=pltpu.SMEM),
            pl.BlockSpec(memory_space=pl.ANY),
        ],
        out_specs=pl.BlockSpec(memory_space=pltpu.SMEM),
        out_shape=jax.ShapeDtypeStruct((1,), jnp.int32),
        scratch_shapes=[
            pltpu.VMEM((_K,), jnp.float32),
            pltpu.VMEM((6, _K), jnp.float32),
            pltpu.SemaphoreType.DMA,
        ],
    )(z_flat, codebook, cur_arr, adjacency)
    return out[0]


def kernel(z_flat, codebook, adjacency, current_sym):
    cur_arr = jnp.asarray(current_sym, dtype=jnp.int32).reshape(1)
    return _run(z_flat, codebook, adjacency, cur_arr)


# speedup vs baseline: 2.3118x; 1.6106x over previous
import jax
import jax.numpy as jnp
from jax import lax
from jax.experimental import pallas as pl
from jax.experimental.pallas import tpu as pltpu

_K = 8192
_D = 256
_NB = 4
_BLK = _K // _NB
_NEG = float(-3e38)
_BIG = float(3e38)


def _body(z_ref, cb_ref, cur_ref, adj_ref, out_ref, dist_ref, rows_ref, sem):
    i = pl.program_id(0)
    z = z_ref[...]
    cb = cb_ref[...]
    z2 = z.reshape(_D, 1)
    ones2 = jnp.ones((_D, 1), jnp.float32)
    a = lax.dot_general(cb, z2, (((1,), (0,)), ((), ())),
                        preferred_element_type=jnp.float32)
    b = lax.dot_general(cb * cb, ones2, (((1,), (0,)), ((), ())),
                        preferred_element_type=jnp.float32)
    dist_ref[pl.ds(i * _BLK, _BLK)] = (b - 2.0 * a).reshape(_BLK)

    @pl.when(i == _NB - 1)
    def _tail():
        d2 = dist_ref[...].reshape(64, 128)
        iota2 = lax.broadcasted_iota(jnp.int32, (64, 128), 0) * 128 + \
            lax.broadcasted_iota(jnp.int32, (64, 128), 1)

        cands = []
        cand_dists = []
        for _ in range(5):
            mn = jnp.min(d2)
            idx = jnp.min(jnp.where(d2 == mn, iota2, jnp.int32(_K)))
            cands.append(idx)
            cand_dists.append(mn)
            d2 = jnp.where(iota2 == idx, _BIG, d2)

        cur = cur_ref[0]
        copies = []
        for r in range(5):
            copies.append(pltpu.make_async_copy(
                adj_ref.at[pl.ds(cands[r], 1)], rows_ref.at[pl.ds(r, 1)],
                sem))
        copies.append(pltpu.make_async_copy(
            adj_ref.at[pl.ds(cur, 1)], rows_ref.at[pl.ds(5, 1)], sem))
        for cpy in copies:
            cpy.start()
        for cpy in copies:
            cpy.wait()

        rows = rows_ref[...]
        gdiff = jnp.mean(jnp.abs(rows[:5, :] - rows[5:6, :]), axis=1)

        best_score = jnp.full((), _NEG, jnp.float32)
        best_s = jnp.int32(0)
        for r in range(5):
            sc = -cand_dists[r] + 0.1 * gdiff[r]
            sc = jnp.where(cands[r] == cur, _NEG, sc)
            take = sc > best_score
            best_score = jnp.where(take, sc, best_score)
            best_s = jnp.where(take, cands[r], best_s)
        out_ref[0] = best_s


@jax.jit
def _run(z_flat, codebook, adjacency, cur_arr):
    out = pl.pallas_call(
        _body,
        grid=(_NB,),
        in_specs=[
            pl.BlockSpec((_D,), lambda i: (0,)),
            pl.BlockSpec((_BLK, _D), lambda i: (i, 0)),
            pl.BlockSpec(memory_space=pltpu.SMEM),
            pl.BlockSpec(memory_space=pl.ANY),
        ],
        out_specs=pl.BlockSpec(memory_space=pltpu.SMEM),
        out_shape=jax.ShapeDtypeStruct((1,), jnp.int32),
        scratch_shapes=[
            pltpu.VMEM((_K,), jnp.float32),
            pltpu.VMEM((6, _K), jnp.float32),
            pltpu.SemaphoreType.DMA,
        ],
    )(z_flat, codebook, cur_arr, adjacency)
    return out[0]


def kernel(z_flat, codebook, adjacency, current_sym):
    cur_arr = jnp.asarray(current_sym, dtype=jnp.int32).reshape(1)
    return _run(z_flat, codebook, adjacency, cur_arr)
